# baseline (speedup 1.0000x reference)
import functools

import jax
import jax.numpy as jnp
from jax import lax
from jax.experimental import pallas as pl
from jax.experimental.pallas import tpu as pltpu
from jax.experimental.pallas import tpu_sc as plsc

BATCH = 16384
DIM = 64
LANES = 16
NUM_CORES = 2
NUM_SUBCORES = 16
NUM_WORKERS = NUM_CORES * NUM_SUBCORES
SW = 128
SID_SH = 7
G8 = 8
N_GROUPS = BATCH // LANES
UEMB_ROWS = BATCH + LANES
DUMMY_PV = 255 << 21

NU = 100000
NI = 1000000
NFU = NU // SW
NFI = NI // SW
TU = NU - NFU * SW
TI = NI - NFI * SW
NGP_U = (NFU // NUM_SUBCORES + 1 + G8 - 1) // G8
NGP_I = (NFI // NUM_WORKERS + 1 + G8 - 1) // G8


def _body(uids_hbm, iids_hbm, ut_hbm, it_hbm, tu_hbm, ti_hbm,
          out2_hbm, uemb_hbm, part_hbm,
          ids, hits1, slab, dimidx, ustage, tu_v, ti_v,
          out_v, row_buf, acc_v, sem):
    c = lax.axis_index("c")
    s = lax.axis_index("s")
    w = s * NUM_CORES + c
    lane = lax.iota(jnp.int32, LANES)
    uemb_base = c * UEMB_ROWS

    for k in range(DIM // LANES):
        dimidx[pl.ds(k * LANES, LANES)] = k * LANES + lane
    pltpu.sync_copy(tu_hbm, tu_v)
    pltpu.sync_copy(ti_hbm, ti_v)

    def scan_hits(owner, stride, ls_shift):
        def g_body(g, cnt):
            idg = ids[pl.ds(g * LANES, LANES)]
            sid = jnp.right_shift(idg, SID_SH)
            mask = jnp.bitwise_and(sid, stride - 1) == owner
            pv = (jnp.left_shift(jnp.right_shift(sid, ls_shift), 21)
                  | jnp.left_shift(jnp.bitwise_and(idg, SW - 1), 14)
                  | (g * LANES + lane))
            plsc.store_compressed(hits1.at[pl.ds(cnt, LANES)], pv, mask=mask)
            return cnt + plsc.all_reduce_population_count(mask)[0]

        cnt = lax.fori_loop(0, N_GROUPS, g_body, 0)
        hits1[pl.ds(cnt, LANES)] = jnp.full((LANES,), DUMMY_PV, jnp.int32)
        return jnp.right_shift(cnt + LANES - 1, 4)

    def resolve(tab_vals, posv, use_m, phase):
        pos_flat = uemb_base + jnp.where(use_m, posv, BATCH + lane)
        if phase == 1:
            for d in range(DIM):
                dv = jnp.full((LANES,), d, jnp.int32)
                plsc.store_scatter(ustage, [lane, dv], tab_vals(d))
            pltpu.async_copy(ustage, uemb_hbm.at[pos_flat], sem).wait()
        else:
            pltpu.async_copy(uemb_hbm.at[pos_flat], ustage, sem).wait()
            acc = jnp.zeros((LANES,), jnp.float32)
            for d in range(DIM):
                dv = jnp.full((LANES,), d, jnp.int32)
                acc = acc + plsc.load_gather(ustage, [lane, dv]) * tab_vals(d)
            plsc.store_scatter(out_v, [posv], acc, mask=use_m)

    def run_phase(tab_hbm, n_full, owner, stride, ls_shift, n_gpass,
                  tail_v, phase):
        n1g = scan_hits(owner, stride, ls_shift)

        def gpass_body(gp, _):
            handles = []
            for j in range(G8):
                t = owner + stride * (gp * G8 + j)
                teff = jnp.minimum(t, n_full - 1)
                base = pl.multiple_of(teff * SW, SW)
                handles.append(pltpu.async_copy(
                    tab_hbm.at[dimidx, pl.ds(base, SW)],
                    slab.at[pl.ds(j * DIM, DIM)], sem))
            for h in handles:
                h.wait()

            def g2_body(g2, _2):
                pv = hits1[pl.ds(g2 * LANES, LANES)]
                kv = jnp.right_shift(pv, 21)
                tail_l = (owner + stride * kv) == n_full
                slab_m = jnp.logical_and(
                    jnp.right_shift(kv, 3) == gp,
                    jnp.logical_not(tail_l))
                pop = plsc.all_reduce_population_count(slab_m)[0]

                @pl.when(pop > 0)
                def _3():
                    offc = jnp.where(
                        slab_m,
                        jnp.bitwise_and(jnp.right_shift(pv, 14), SW - 1), 0)
                    rowb = jnp.where(
                        slab_m, jnp.bitwise_and(kv, G8 - 1) * DIM, 0)
                    posv = jnp.bitwise_and(pv, BATCH - 1)
                    resolve(lambda d: plsc.load_gather(
                        slab, [rowb + d, offc]), posv, slab_m, phase)
                return 0

            lax.fori_loop(0, n1g, g2_body, 0)
            return 0

        lax.fori_loop(0, n_gpass, gpass_body, 0)

        def tail_body(g2, _2):
            pv = hits1[pl.ds(g2 * LANES, LANES)]
            kv = jnp.right_shift(pv, 21)
            tail_m = (owner + stride * kv) == n_full
            pop = plsc.all_reduce_population_count(tail_m)[0]

            @pl.when(pop > 0)
            def _3():
                idrel = jnp.where(
                    tail_m,
                    jnp.bitwise_and(jnp.right_shift(pv, 14), SW - 1), 0)
                posv = jnp.bitwise_and(pv, BATCH - 1)
                resolve(lambda d: plsc.load_gather(
                    tail_v, [idrel, jnp.full((LANES,), d, jnp.int32)]),
                    posv, tail_m, phase)
            return 0

        lax.fori_loop(0, n1g, tail_body, 0)

    pltpu.sync_copy(uids_hbm, ids)
    run_phase(ut_hbm, NFU, s, NUM_SUBCORES, 4, NGP_U, tu_v, 1)
    plsc.subcore_barrier()

    def zero_body(g, _):
        out_v[pl.ds(g * LANES, LANES)] = jnp.zeros((LANES,), jnp.float32)
        return 0
    lax.fori_loop(0, N_GROUPS, zero_body, 0)
    pltpu.sync_copy(iids_hbm, ids)
    run_phase(it_hbm, NFI, w, NUM_WORKERS, 5, NGP_I, ti_v, 2)

    pltpu.sync_copy(out_v, part_hbm.at[pl.ds((c * NUM_SUBCORES + s) * BATCH,
                                             BATCH)])
    plsc.subcore_barrier()
    for half in range(2):
        col0 = s * 1024 + half * 512

        def zacc_body(g, _):
            acc_v[pl.ds(g * LANES, LANES)] = jnp.zeros((LANES,), jnp.float32)
            return 0
        lax.fori_loop(0, 512 // LANES, zacc_body, 0)
        for r in range(NUM_SUBCORES):
            pltpu.sync_copy(
                part_hbm.at[pl.ds((c * NUM_SUBCORES + r) * BATCH + col0,
                                  512)], row_buf)

            def add_body(g, _):
                sl = pl.ds(g * LANES, LANES)
                acc_v[sl] = acc_v[sl] + row_buf[sl]
                return 0
            lax.fori_loop(0, 512 // LANES, add_body, 0)
        pltpu.sync_copy(acc_v, out2_hbm.at[pl.ds(c * BATCH + col0, 512)])


def kernel(user_ids, item_ids, user_table, item_table):
    ut = user_table.T
    it = item_table.T
    tail_u = jnp.pad(user_table[NFU * SW:], ((0, 0), (0, SW - DIM)))
    tail_i = jnp.pad(item_table[NFI * SW:], ((0, 0), (0, SW - DIM)))
    uids = user_ids.astype(jnp.int32)
    iids = item_ids.astype(jnp.int32)

    mesh = plsc.VectorSubcoreMesh(
        core_axis_name="c", subcore_axis_name="s",
        num_cores=NUM_CORES, num_subcores=NUM_SUBCORES)

    run = pl.kernel(
        _body,
        out_type=[
            jax.ShapeDtypeStruct((NUM_CORES * BATCH,), jnp.float32),
            jax.ShapeDtypeStruct((NUM_CORES * UEMB_ROWS, SW), jnp.float32),
            jax.ShapeDtypeStruct((NUM_WORKERS * BATCH,), jnp.float32),
        ],
        mesh=mesh,
        scratch_types=[
            pltpu.VMEM((BATCH,), jnp.int32),
            pltpu.VMEM((BATCH + LANES,), jnp.int32),
            pltpu.VMEM((G8 * DIM, SW), jnp.float32),
            pltpu.VMEM((DIM,), jnp.int32),
            pltpu.VMEM((LANES, SW), jnp.float32),
            pltpu.VMEM((TU, SW), jnp.float32),
            pltpu.VMEM((TI, SW), jnp.float32),
            pltpu.VMEM((BATCH,), jnp.float32),
            pltpu.VMEM((512,), jnp.float32),
            pltpu.VMEM((512,), jnp.float32),
            pltpu.SemaphoreType.DMA,
        ],
        compiler_params=pltpu.CompilerParams(
            needs_layout_passes=False, use_tc_tiling_on_sc=True),
    )
    out2, _, _ = run(uids, iids, ut, it, tail_u, tail_i)
    return out2[:BATCH] + out2[BATCH:]
